# initial kernel scaffold (unmeasured)
import jax
import jax.numpy as jnp
from jax import lax
from jax.experimental import pallas as pl
from jax.experimental.pallas import tpu as pltpu

N_DEV = 32
B, SQ, SKV, DH = 2, 128, 128, 64
H_PER = 4
D_MODEL = 512
ROWS = B * SQ
CHUNK = ROWS // N_DEV


def _body(x_ref, wq_ref, k_ref, v_ref, wo_ref, out_ref,
          p_ref, rs_ref,
          p_send_sems, p_recv_sems, g_send_sems, g_recv_sems):
    my_i = lax.axis_index("i")

    q = jnp.dot(x_ref[...], wq_ref[...],
                preferred_element_type=jnp.float32)

    qb = lax.broadcasted_iota(jnp.int32, (SQ, SKV), 0) // 64
    kb = lax.broadcasted_iota(jnp.int32, (SQ, SKV), 1) // 64
    mask = (qb == kb) | (kb == 0) | (((qb + kb) % 3) == 0)

    rows = []
    for b in range(B):
        heads = []
        for h in range(H_PER):
            qbh = q[b * SQ:(b + 1) * SQ, h * DH:(h + 1) * DH]
            base = (b * H_PER + h) * SKV
            kbh = k_ref[base:base + SKV, :]
            vbh = v_ref[base:base + SKV, :]
            s = lax.dot_general(qbh, kbh, (((1,), (1,)), ((), ())),
                                preferred_element_type=jnp.float32) * 0.125
            s = jnp.where(mask, s, -1e9)
            w = jnp.exp(s - jnp.max(s, axis=-1, keepdims=True))
            w = w / jnp.sum(w, axis=-1, keepdims=True)
            heads.append(jnp.dot(w, vbh, preferred_element_type=jnp.float32))
        rows.append(jnp.concatenate(heads, axis=1))
    ctx = jnp.concatenate(rows, axis=0)
    p_ref[...] = jnp.dot(ctx, wo_ref[...],
                         preferred_element_type=jnp.float32)

    p_sends = []
    for d in range(1, N_DEV):
        t = lax.rem(my_i + d, N_DEV)
        rdma = pltpu.make_async_remote_copy(
            src_ref=p_ref.at[pl.ds(t * CHUNK, CHUNK)],
            dst_ref=rs_ref.at[my_i],
            send_sem=p_send_sems.at[d - 1],
            recv_sem=p_recv_sems.at[my_i],
            device_id=(t,),
            device_id_type=pl.DeviceIdType.MESH,
        )
        rdma.start()
        p_sends.append(rdma)

    rs_ref[pl.ds(my_i, 1)] = p_ref[pl.ds(my_i * CHUNK, CHUNK), :][None]

    for d in range(1, N_DEV):
        s = lax.rem(my_i - d + N_DEV, N_DEV)
        recv = pltpu.make_async_remote_copy(
            src_ref=rs_ref.at[s],
            dst_ref=rs_ref.at[s],
            send_sem=p_send_sems.at[d - 1],
            recv_sem=p_recv_sems.at[s],
            device_id=(s,),
            device_id_type=pl.DeviceIdType.MESH,
        )
        recv.wait_recv()
    for rdma in p_sends:
        rdma.wait_send()

    out_ref[pl.ds(my_i * CHUNK, CHUNK), :] = jnp.sum(rs_ref[...], axis=0)

    g_sends = []
    for d in range(1, N_DEV):
        t = lax.rem(my_i + d, N_DEV)
        rdma = pltpu.make_async_remote_copy(
            src_ref=out_ref.at[pl.ds(my_i * CHUNK, CHUNK)],
            dst_ref=out_ref.at[pl.ds(my_i * CHUNK, CHUNK)],
            send_sem=g_send_sems.at[d - 1],
            recv_sem=g_recv_sems.at[my_i],
            device_id=(t,),
            device_id_type=pl.DeviceIdType.MESH,
        )
        rdma.start()
        g_sends.append(rdma)

    for d in range(1, N_DEV):
        s = lax.rem(my_i - d + N_DEV, N_DEV)
        recv = pltpu.make_async_remote_copy(
            src_ref=out_ref.at[pl.ds(s * CHUNK, CHUNK)],
            dst_ref=out_ref.at[pl.ds(s * CHUNK, CHUNK)],
            send_sem=g_send_sems.at[d - 1],
            recv_sem=g_recv_sems.at[s],
            device_id=(s,),
            device_id_type=pl.DeviceIdType.MESH,
        )
        recv.wait_recv()
    for rdma in g_sends:
        rdma.wait_send()


def kernel(x, Wq, K_ext, V_ext, Wo):
    i = lax.axis_index("i")
    Ks = lax.dynamic_slice_in_dim(K_ext, i * H_PER, H_PER, axis=2)
    Vs = lax.dynamic_slice_in_dim(V_ext, i * H_PER, H_PER, axis=2)
    k2 = jnp.transpose(Ks, (0, 2, 1, 3)).reshape(B * H_PER * SKV, DH)
    v2 = jnp.transpose(Vs, (0, 2, 1, 3)).reshape(B * H_PER * SKV, DH)
    xf = x.reshape(ROWS, D_MODEL)

    out = pl.pallas_call(
        _body,
        out_shape=jax.ShapeDtypeStruct((ROWS, D_MODEL), jnp.float32),
        in_specs=[pl.BlockSpec(memory_space=pltpu.VMEM)] * 5,
        out_specs=pl.BlockSpec(memory_space=pltpu.VMEM),
        scratch_shapes=[
            pltpu.VMEM((ROWS, D_MODEL), jnp.float32),
            pltpu.VMEM((N_DEV, CHUNK, D_MODEL), jnp.float32),
            pltpu.SemaphoreType.DMA((N_DEV,)),
            pltpu.SemaphoreType.DMA((N_DEV,)),
            pltpu.SemaphoreType.DMA((N_DEV,)),
            pltpu.SemaphoreType.DMA((N_DEV,)),
        ],
        compiler_params=pltpu.CompilerParams(collective_id=0),
    )(xf, Wq, k2, v2, Wo)
    return out.reshape(B, SQ, D_MODEL)


# baseline (device time: 48345 ns/iter reference)
import jax
import jax.numpy as jnp
from jax import lax
from jax.experimental import pallas as pl
from jax.experimental.pallas import tpu as pltpu

N_DEV = 32
B, SQ, SKV, DH = 2, 128, 128, 64
H_PER = 4
D_MODEL = 512
ROWS = B * SQ
CHUNK = ROWS // N_DEV


def _body(x_ref, wq_ref, k_ref, v_ref, wo_ref, out_ref,
          p_ref, rs_ref,
          p_send_sems, p_recv_sems, g_send_sems, g_recv_sems):
    my_i = lax.axis_index("i")

    q = jnp.dot(x_ref[...], wq_ref[...],
                preferred_element_type=jnp.float32)

    qb = lax.broadcasted_iota(jnp.int32, (SQ, SKV), 0) // 64
    kb = lax.broadcasted_iota(jnp.int32, (SQ, SKV), 1) // 64
    mask = (qb == kb) | (kb == 0) | (((qb + kb) % 3) == 0)

    rows = []
    for b in range(B):
        heads = []
        for h in range(H_PER):
            qbh = q[b * SQ:(b + 1) * SQ, h * DH:(h + 1) * DH]
            base = (b * H_PER + h) * SKV
            kbh = k_ref[base:base + SKV, :]
            vbh = v_ref[base:base + SKV, :]
            s = lax.dot_general(qbh, kbh, (((1,), (1,)), ((), ())),
                                preferred_element_type=jnp.float32) * 0.125
            s = jnp.where(mask, s, -1e9)
            w = jnp.exp(s - jnp.max(s, axis=-1, keepdims=True))
            w = w / jnp.sum(w, axis=-1, keepdims=True)
            heads.append(jnp.dot(w, vbh, preferred_element_type=jnp.float32))
        rows.append(jnp.concatenate(heads, axis=1))
    ctx = jnp.concatenate(rows, axis=0)
    p_ref[...] = jnp.dot(ctx, wo_ref[...],
                         preferred_element_type=jnp.float32)

    p_sends = []
    for d in range(1, N_DEV):
        t = lax.rem(my_i + d, N_DEV)
        rdma = pltpu.make_async_remote_copy(
            src_ref=p_ref.at[pl.ds(t * CHUNK, CHUNK)],
            dst_ref=rs_ref.at[my_i],
            send_sem=p_send_sems.at[d - 1],
            recv_sem=p_recv_sems.at[my_i],
            device_id=(t,),
            device_id_type=pl.DeviceIdType.MESH,
        )
        rdma.start()
        p_sends.append(rdma)

    rs_ref[pl.ds(my_i, 1)] = p_ref[pl.ds(my_i * CHUNK, CHUNK), :][None]

    for d in range(1, N_DEV):
        s = lax.rem(my_i - d + N_DEV, N_DEV)
        recv = pltpu.make_async_remote_copy(
            src_ref=rs_ref.at[s],
            dst_ref=rs_ref.at[s],
            send_sem=p_send_sems.at[d - 1],
            recv_sem=p_recv_sems.at[s],
            device_id=(s,),
            device_id_type=pl.DeviceIdType.MESH,
        )
        recv.wait_recv()
    for rdma in p_sends:
        rdma.wait_send()

    out_ref[pl.ds(my_i * CHUNK, CHUNK), :] = jnp.sum(rs_ref[...], axis=0)

    g_sends = []
    for d in range(1, N_DEV):
        t = lax.rem(my_i + d, N_DEV)
        rdma = pltpu.make_async_remote_copy(
            src_ref=out_ref.at[pl.ds(my_i * CHUNK, CHUNK)],
            dst_ref=out_ref.at[pl.ds(my_i * CHUNK, CHUNK)],
            send_sem=g_send_sems.at[d - 1],
            recv_sem=g_recv_sems.at[my_i],
            device_id=(t,),
            device_id_type=pl.DeviceIdType.MESH,
        )
        rdma.start()
        g_sends.append(rdma)

    for d in range(1, N_DEV):
        s = lax.rem(my_i - d + N_DEV, N_DEV)
        recv = pltpu.make_async_remote_copy(
            src_ref=out_ref.at[pl.ds(s * CHUNK, CHUNK)],
            dst_ref=out_ref.at[pl.ds(s * CHUNK, CHUNK)],
            send_sem=g_send_sems.at[d - 1],
            recv_sem=g_recv_sems.at[s],
            device_id=(s,),
            device_id_type=pl.DeviceIdType.MESH,
        )
        recv.wait_recv()
    for rdma in g_sends:
        rdma.wait_send()


def kernel(x, Wq, K_ext, V_ext, Wo):
    i = lax.axis_index("i")
    Ks = lax.dynamic_slice_in_dim(K_ext, i * H_PER, H_PER, axis=2)
    Vs = lax.dynamic_slice_in_dim(V_ext, i * H_PER, H_PER, axis=2)
    k2 = jnp.transpose(Ks, (0, 2, 1, 3)).reshape(B * H_PER * SKV, DH)
    v2 = jnp.transpose(Vs, (0, 2, 1, 3)).reshape(B * H_PER * SKV, DH)
    xf = x.reshape(ROWS, D_MODEL)

    out = pl.pallas_call(
        _body,
        out_shape=jax.ShapeDtypeStruct((ROWS, D_MODEL), jnp.float32),
        in_specs=[pl.BlockSpec(memory_space=pltpu.VMEM)] * 5,
        out_specs=pl.BlockSpec(memory_space=pltpu.VMEM),
        scratch_shapes=[
            pltpu.VMEM((ROWS, D_MODEL), jnp.float32),
            pltpu.VMEM((N_DEV, CHUNK, D_MODEL), jnp.float32),
            pltpu.SemaphoreType.DMA((N_DEV,)),
            pltpu.SemaphoreType.DMA((N_DEV,)),
            pltpu.SemaphoreType.DMA((N_DEV,)),
            pltpu.SemaphoreType.DMA((N_DEV,)),
        ],
    )(xf, Wq, k2, v2, Wo)
    return out.reshape(B, SQ, D_MODEL)


# device time: 39879 ns/iter; 1.2123x vs baseline; 1.2123x over previous
import jax
import jax.numpy as jnp
from jax import lax
from jax.experimental import pallas as pl
from jax.experimental.pallas import tpu as pltpu

N_DEV = 32
B, SQ, SKV, DH = 2, 128, 128, 64
H_PER = 4
D_MODEL = 512
ROWS = B * SQ
CHUNK = ROWS // N_DEV

DO_COMPUTE = True
DO_COMM = True


def _comm(out_ref, pb_ref, rs_ref, gb_ref,
          p_send_sems, p_recv_sems, g_send_sems, g_recv_sems,
          barrier_sem, my_i):
    pl.semaphore_wait(barrier_sem, N_DEV - 1)

    p_sends = []
    for d in range(1, N_DEV):
        t = lax.rem(my_i + d, N_DEV)
        rdma = pltpu.make_async_remote_copy(
            src_ref=pb_ref.at[pl.ds(t * CHUNK, CHUNK)],
            dst_ref=rs_ref.at[my_i],
            send_sem=p_send_sems.at[d - 1],
            recv_sem=p_recv_sems.at[my_i],
            device_id=(t,),
            device_id_type=pl.DeviceIdType.MESH,
        )
        rdma.start()
        p_sends.append(rdma)

    rs_ref[pl.ds(my_i, 1)] = pb_ref[pl.ds(my_i * CHUNK, CHUNK), :][None]

    for d in range(1, N_DEV):
        s = lax.rem(my_i - d + N_DEV, N_DEV)
        recv = pltpu.make_async_remote_copy(
            src_ref=rs_ref.at[s],
            dst_ref=rs_ref.at[s],
            send_sem=p_send_sems.at[d - 1],
            recv_sem=p_recv_sems.at[s],
            device_id=(s,),
            device_id_type=pl.DeviceIdType.MESH,
        )
        recv.wait_recv()
    for rdma in p_sends:
        rdma.wait_send()

    red = jnp.sum(rs_ref[...].astype(jnp.float32), axis=0)
    gb_ref[pl.ds(my_i * CHUNK, CHUNK), :] = red.astype(jnp.bfloat16)

    g_sends = []
    for d in range(1, N_DEV):
        t = lax.rem(my_i + d, N_DEV)
        rdma = pltpu.make_async_remote_copy(
            src_ref=gb_ref.at[pl.ds(my_i * CHUNK, CHUNK)],
            dst_ref=gb_ref.at[pl.ds(my_i * CHUNK, CHUNK)],
            send_sem=g_send_sems.at[d - 1],
            recv_sem=g_recv_sems.at[my_i],
            device_id=(t,),
            device_id_type=pl.DeviceIdType.MESH,
        )
        rdma.start()
        g_sends.append(rdma)

    for d in range(1, N_DEV):
        s = lax.rem(my_i - d + N_DEV, N_DEV)
        recv = pltpu.make_async_remote_copy(
            src_ref=gb_ref.at[pl.ds(s * CHUNK, CHUNK)],
            dst_ref=gb_ref.at[pl.ds(s * CHUNK, CHUNK)],
            send_sem=g_send_sems.at[d - 1],
            recv_sem=g_recv_sems.at[s],
            device_id=(s,),
            device_id_type=pl.DeviceIdType.MESH,
        )
        recv.wait_recv()

    out_ref[...] = gb_ref[...].astype(jnp.float32)

    for rdma in g_sends:
        rdma.wait_send()


def _body(x_ref, wq_ref, k_hbm, v_hbm, wo_ref, out_ref,
          k_vmem, v_vmem, pb_ref, rs_ref, gb_ref,
          kv_sems, p_send_sems, p_recv_sems, g_send_sems, g_recv_sems):
    my_i = lax.axis_index("i")

    barrier_sem = None
    if DO_COMM:
        barrier_sem = pltpu.get_barrier_semaphore()
        for d in range(1, N_DEV):
            t = lax.rem(my_i + d, N_DEV)
            pl.semaphore_signal(
                barrier_sem, inc=1,
                device_id=(t,), device_id_type=pl.DeviceIdType.MESH,
            )

    if DO_COMPUTE:
        h0 = my_i * H_PER
        copies = []
        for b in range(B):
            for h in range(H_PER):
                idx = b * H_PER + h
                c = pltpu.make_async_copy(
                    k_hbm.at[b, :, h0 + h, :], k_vmem.at[b, h],
                    kv_sems.at[idx])
                c.start()
                copies.append(c)
                c = pltpu.make_async_copy(
                    v_hbm.at[b, :, h0 + h, :], v_vmem.at[b, h],
                    kv_sems.at[B * H_PER + idx])
                c.start()
                copies.append(c)

        q = jnp.dot(x_ref[...], wq_ref[...],
                    preferred_element_type=jnp.float32)

        qb = lax.broadcasted_iota(jnp.int32, (SQ, SKV), 0) // 64
        kb = lax.broadcasted_iota(jnp.int32, (SQ, SKV), 1) // 64
        mask = (qb == kb) | (kb == 0) | (((qb + kb) % 3) == 0)

        for c in copies:
            c.wait()

        rows = []
        for b in range(B):
            heads = []
            for h in range(H_PER):
                qbh = q[b * SQ:(b + 1) * SQ, h * DH:(h + 1) * DH]
                kbh = k_vmem[b, h]
                vbh = v_vmem[b, h]
                s = lax.dot_general(qbh, kbh, (((1,), (1,)), ((), ())),
                                    preferred_element_type=jnp.float32) * 0.125
                s = jnp.where(mask, s, -1e9)
                w = jnp.exp(s - jnp.max(s, axis=-1, keepdims=True))
                w = w / jnp.sum(w, axis=-1, keepdims=True)
                heads.append(jnp.dot(w, vbh,
                                     preferred_element_type=jnp.float32))
            rows.append(jnp.concatenate(heads, axis=1))
        ctx = jnp.concatenate(rows, axis=0)
        p = jnp.dot(ctx, wo_ref[...],
                    preferred_element_type=jnp.float32)
    else:
        p = x_ref[...]

    if not DO_COMM:
        out_ref[...] = p
        return

    pb_ref[...] = p.astype(jnp.bfloat16)
    _comm(out_ref, pb_ref, rs_ref, gb_ref,
          p_send_sems, p_recv_sems, g_send_sems, g_recv_sems,
          barrier_sem, my_i)


def kernel(x, Wq, K_ext, V_ext, Wo):
    xf = x.reshape(ROWS, D_MODEL)

    params = {}
    if DO_COMM:
        params["compiler_params"] = pltpu.CompilerParams(collective_id=0)

    out = pl.pallas_call(
        _body,
        out_shape=jax.ShapeDtypeStruct((ROWS, D_MODEL), jnp.float32),
        in_specs=[
            pl.BlockSpec(memory_space=pltpu.VMEM),
            pl.BlockSpec(memory_space=pltpu.VMEM),
            pl.BlockSpec(memory_space=pl.ANY),
            pl.BlockSpec(memory_space=pl.ANY),
            pl.BlockSpec(memory_space=pltpu.VMEM),
        ],
        out_specs=pl.BlockSpec(memory_space=pltpu.VMEM),
        scratch_shapes=[
            pltpu.VMEM((B, H_PER, SKV, DH), jnp.float32),
            pltpu.VMEM((B, H_PER, SKV, DH), jnp.float32),
            pltpu.VMEM((ROWS, D_MODEL), jnp.bfloat16),
            pltpu.VMEM((N_DEV, CHUNK, D_MODEL), jnp.bfloat16),
            pltpu.VMEM((ROWS, D_MODEL), jnp.bfloat16),
            pltpu.SemaphoreType.DMA((2 * B * H_PER,)),
            pltpu.SemaphoreType.DMA((N_DEV,)),
            pltpu.SemaphoreType.DMA((N_DEV,)),
            pltpu.SemaphoreType.DMA((N_DEV,)),
            pltpu.SemaphoreType.DMA((N_DEV,)),
        ],
        **params,
    )(xf, Wq, K_ext, V_ext, Wo)
    return out.reshape(B, SQ, D_MODEL)
